# baseline (speedup 1.0000x reference)
import jax
import jax.numpy as jnp
from jax import lax
from jax.experimental import pallas as pl
from jax.experimental.pallas import tpu as pltpu
from jax.experimental.pallas import tpu_sc as plsc

_N = 10000
_E = 320000
_H = 128
_LATENT = 256
_NRBF = 16
_MAXZ = 100
_CUTOFF = 6.0
_LANES = 16

_NC, _NS = 2, 16
_NW = _NC * _NS
_EW = _E // _NW
_K = 80
_NCHUNK = _EW // _K
_RPT = _N // _NS

_BN = 2000
_BE = 2560

_f32 = jnp.float32
_bf16 = jnp.bfloat16


def _dot_bf16(a, b):
    return jnp.dot(a.astype(_bf16), b.astype(_bf16), preferred_element_type=_f32)


def _cart_body(pf_ref, ln_ref, an_ref, out_ref):
    pf = pf_ref[...]
    f = pf - jnp.floor(pf)
    ln = ln_ref[...]
    rad = an_ref[...] * (jnp.pi / 180.0)
    ca = jnp.cos(rad[:, 0:1])
    cb = jnp.cos(rad[:, 1:2])
    cg = jnp.cos(rad[:, 2:3])
    sg = jnp.sin(rad[:, 2:3])
    a = ln[:, 0:1]
    b = ln[:, 1:2]
    c = ln[:, 2:3]
    cx = cb
    cy = (ca - cb * cg) / sg
    cz = jnp.sqrt(jnp.maximum(1.0 - cx * cx - cy * cy, 1e-6))
    tb = lambda x: x.astype(_bf16).astype(_f32)
    f0, f1, f2 = tb(f[:, 0:1]), tb(f[:, 1:2]), tb(f[:, 2:3])
    l00, l10, l20 = tb(a), tb(b * cg), tb(c * cx)
    l11, l21 = tb(b * sg), tb(c * cy)
    l22 = tb(c * cz)
    cartx = (f0 * l00 + f1 * l10) + f2 * l20
    carty = f1 * l11 + f2 * l21
    cartz = f2 * l22
    pad = jnp.zeros((pf.shape[0], _LANES - 3), _f32)
    out_ref[...] = jnp.concatenate([cartx, carty, cartz, pad], axis=1)


_cart_call = pl.pallas_call(
    _cart_body,
    grid=(_N // _BN,),
    in_specs=[pl.BlockSpec((_BN, 3), lambda i: (i, 0))] * 3,
    out_specs=pl.BlockSpec((_BN, _LANES), lambda i: (i, 0)),
    out_shape=jax.ShapeDtypeStruct((_N, _LANES), _f32),
)


def _geom_body(v_ref, wr_ref, bm_ref, cen_ref, u_ref, r_ref):
    v = v_ref[...]
    d2 = jnp.sum(v * v, axis=1, keepdims=True)
    dist = jnp.sqrt(d2) + 1e-8
    u_ref[...] = v / dist
    rbf = jnp.exp(-2.0 * (dist - cen_ref[...]) ** 2)
    r_ref[...] = _dot_bf16(rbf, wr_ref[...]) + bm_ref[...]


_geom_call = pl.pallas_call(
    _geom_body,
    grid=(_E // _BE,),
    in_specs=[
        pl.BlockSpec((_BE, _LANES), lambda i: (i, 0)),
        pl.BlockSpec((_NRBF, _H), lambda i: (0, 0)),
        pl.BlockSpec((1, _H), lambda i: (0, 0)),
        pl.BlockSpec((1, _NRBF), lambda i: (0, 0)),
    ],
    out_specs=[
        pl.BlockSpec((_BE, _LANES), lambda i: (i, 0)),
        pl.BlockSpec((_BE, _H), lambda i: (i, 0)),
    ],
    out_shape=[
        jax.ShapeDtypeStruct((_E, _LANES), _f32),
        jax.ShapeDtypeStruct((_E, _H), _f32),
    ],
)


def _proj_body(h_ref, ws_ref, wd_ref, hs_ref, hd_ref):
    h = h_ref[...]
    hs_ref[...] = _dot_bf16(h, ws_ref[...])
    hd_ref[...] = _dot_bf16(h, wd_ref[...])


_proj_call = pl.pallas_call(
    _proj_body,
    grid=(_N // _BN,),
    in_specs=[
        pl.BlockSpec((_BN, _H), lambda i: (i, 0)),
        pl.BlockSpec((_H, _H), lambda i: (0, 0)),
        pl.BlockSpec((_H, _H), lambda i: (0, 0)),
    ],
    out_specs=[pl.BlockSpec((_BN, _H), lambda i: (i, 0))] * 2,
    out_shape=[jax.ShapeDtypeStruct((_N, _H), _f32)] * 2,
)


def _update_body(h_ref, a0_ref, a1_ref, wu_ref, bu_ref, ws_ref, wd_ref,
                 hn_ref, hs_ref, hd_ref):
    agg = a0_ref[...] + a1_ref[...]
    hn = h_ref[...] + jnp.maximum(_dot_bf16(agg, wu_ref[...]) + bu_ref[...], 0.0)
    hn_ref[...] = hn
    hs_ref[...] = _dot_bf16(hn, ws_ref[...])
    hd_ref[...] = _dot_bf16(hn, wd_ref[...])


_update_call = pl.pallas_call(
    _update_body,
    grid=(_N // _BN,),
    in_specs=[
        pl.BlockSpec((_BN, _H), lambda i: (i, 0)),
        pl.BlockSpec((_BN, _H), lambda i: (i, 0)),
        pl.BlockSpec((_BN, _H), lambda i: (i, 0)),
        pl.BlockSpec((_H, _H), lambda i: (0, 0)),
        pl.BlockSpec((1, _H), lambda i: (0, 0)),
        pl.BlockSpec((_H, _H), lambda i: (0, 0)),
        pl.BlockSpec((_H, _H), lambda i: (0, 0)),
    ],
    out_specs=[pl.BlockSpec((_BN, _H), lambda i: (i, 0))] * 3,
    out_shape=[jax.ShapeDtypeStruct((_N, _H), _f32)] * 3,
)


def _force_body(m_ref, u_ref, wf_ref, fu_ref):
    mm = m_ref[...].astype(_bf16).astype(_f32)
    ww = wf_ref[...].astype(_bf16).astype(_f32)
    f = jnp.sum(mm * ww, axis=1, keepdims=True)
    fu_ref[...] = f * u_ref[...]


_force_call = pl.pallas_call(
    _force_body,
    grid=(_E // _BE,),
    in_specs=[
        pl.BlockSpec((_BE, _H), lambda i: (i, 0)),
        pl.BlockSpec((_BE, _LANES), lambda i: (i, 0)),
        pl.BlockSpec((1, _H), lambda i: (0, 0)),
    ],
    out_specs=pl.BlockSpec((_BE, _LANES), lambda i: (i, 0)),
    out_shape=jax.ShapeDtypeStruct((_E, _LANES), _f32),
)


def _final_body(h_ref, z_ref, wh_ref, wz_ref, b_ref, f0_ref, f1_ref,
                log_ref, d_ref):
    zz = z_ref[...]
    log_ref[...] = (
        (_dot_bf16(h_ref[...], wh_ref[...]) + _dot_bf16(zz[:, :_H], wz_ref[:_H, :]))
        + _dot_bf16(zz[:, _H:], wz_ref[_H:, :])
    ) + b_ref[...]
    d_ref[...] = f0_ref[...] + f1_ref[...]


_final_call = pl.pallas_call(
    _final_body,
    grid=(_N // _BN,),
    in_specs=[
        pl.BlockSpec((_BN, _H), lambda i: (i, 0)),
        pl.BlockSpec((_BN, _LATENT), lambda i: (i, 0)),
        pl.BlockSpec((_H, _MAXZ), lambda i: (0, 0)),
        pl.BlockSpec((_LATENT, _MAXZ), lambda i: (0, 0)),
        pl.BlockSpec((1, _MAXZ), lambda i: (0, 0)),
        pl.BlockSpec((_BN, _LANES), lambda i: (i, 0)),
        pl.BlockSpec((_BN, _LANES), lambda i: (i, 0)),
    ],
    out_specs=[
        pl.BlockSpec((_BN, _MAXZ), lambda i: (i, 0)),
        pl.BlockSpec((_BN, _LANES), lambda i: (i, 0)),
    ],
    out_shape=[
        jax.ShapeDtypeStruct((_N, _MAXZ), _f32),
        jax.ShapeDtypeStruct((_N, _LANES), _f32),
    ],
)


def _vec_body(cart_hbm, sd_hbm, out_hbm, sd0, sd1, a0, b0, a1, b1,
              sa0, sb0, sa1, sb1, ss0, ss1):
    cid = lax.axis_index("c")
    sid = lax.axis_index("s")
    wid = sid * _NC + cid
    ebase = wid * _EW
    bufs = ((sd0, a0, b0, sa0, sb0, ss0), (sd1, a1, b1, sa1, sb1, ss1))

    def issue(ci, p, drain):
        sd, a, b, sa, sb, ss = bufs[p]

        def _drain():
            pltpu.make_async_copy(a, out_hbm.at[pl.ds(ebase + (ci - 2) * _K2, _K2)], ss).wait()

        if drain is True:
            _drain()
        elif drain is not False:
            pl.when(drain)(_drain)
        pltpu.sync_copy(sd_hbm.at[wid, ci], sd)
        pltpu.async_copy(cart_hbm.at[sd.at[0]], a, sa)
        pltpu.async_copy(cart_hbm.at[sd.at[1]], b, sb)

    def finish(ci, p):
        sd, a, b, sa, sb, ss = bufs[p]
        pltpu.make_async_copy(cart_hbm.at[sd.at[0]], a, sa).wait()
        pltpu.make_async_copy(cart_hbm.at[sd.at[1]], b, sb).wait()

        def row(i, cr):
            a[i, :] = a[i, :] - b[i, :]
            return cr

        lax.fori_loop(0, _K2, row, 0)
        pltpu.async_copy(a, out_hbm.at[pl.ds(ebase + ci * _K2, _K2)], ss)

    issue(0, 0, False)

    def pair(g, carry):
        ci0 = 2 * g
        issue(ci0 + 1, 1, g > 0)
        finish(ci0, 0)

        @pl.when(g < _NCHUNK2 // 2 - 1)
        def _():
            issue(ci0 + 2, 0, True)

        finish(ci0 + 1, 1)
        return carry

    lax.fori_loop(0, _NCHUNK2 // 2, pair, 0)
    for p, ci in ((0, _NCHUNK2 - 2), (1, _NCHUNK2 - 1)):
        sd, a, b, sa, sb, ss = bufs[p]
        pltpu.make_async_copy(a, out_hbm.at[pl.ds(ebase + ci * _K2, _K2)], ss).wait()


def _vec_call_build():
  return pl.kernel(
    _vec_body,
    out_type=jax.ShapeDtypeStruct((_E, _LANES), _f32),
    mesh=plsc.VectorSubcoreMesh(core_axis_name="c", subcore_axis_name="s", num_cores=_NC, num_subcores=_NS),
    compiler_params=pltpu.CompilerParams(use_tc_tiling_on_sc=False),
    scratch_types=[
        pltpu.VMEM((2, _K2), jnp.int32),
        pltpu.VMEM((2, _K2), jnp.int32),
        pltpu.VMEM((_K2, _LANES), _f32),
        pltpu.VMEM((_K2, _LANES), _f32),
        pltpu.VMEM((_K2, _LANES), _f32),
        pltpu.VMEM((_K2, _LANES), _f32),
        pltpu.SemaphoreType.DMA,
        pltpu.SemaphoreType.DMA,
        pltpu.SemaphoreType.DMA,
        pltpu.SemaphoreType.DMA,
        pltpu.SemaphoreType.DMA,
        pltpu.SemaphoreType.DMA,
    ],
  )


_AW = 25
_AK = 80


def _embg_body(emb_hbm, t_hbm, out_hbm, tidx, rbuf, sg):
    cid = lax.axis_index("c")
    sid = lax.axis_index("s")
    wid = sid * _NC + cid
    base = wid * (_AK * 5)

    @pl.when(wid < _AW)
    def _():
        def chunk(ci, carry):
            off = base + ci * _AK
            pltpu.sync_copy(t_hbm.at[pl.ds(off, _AK)], tidx)
            pltpu.async_copy(emb_hbm.at[tidx], rbuf, sg).wait()
            pltpu.sync_copy(rbuf, out_hbm.at[pl.ds(off, _AK)])
            return carry

        lax.fori_loop(0, 5, chunk, 0)


def _embg_call_build():
    return pl.kernel(
        _embg_body,
        out_type=jax.ShapeDtypeStruct((_N, _H), _f32),
        mesh=plsc.VectorSubcoreMesh(core_axis_name="c", subcore_axis_name="s", num_cores=_NC, num_subcores=_NS),
        compiler_params=pltpu.CompilerParams(use_tc_tiling_on_sc=False),
        scratch_types=[
            pltpu.VMEM((_AK,), jnp.int32),
            pltpu.VMEM((_AK, _H), _f32),
            pltpu.SemaphoreType.DMA,
        ],
    )


_K2 = 40
_NCHUNK2 = _EW // _K2


def _make_msgpass(write_m):
    def body(hs_hbm, hd_hbm, rbf_hbm, sd_hbm, zeros_hbm, out_hbm,
             sd0, sd1, a0, b0, c0, a1, b1, c1, sA0, sB0, sC0, sA1,
             sB1, sC1, sS0, sS1, *rest):
        cid = lax.axis_index("c")
        sid = lax.axis_index("s")
        wid = sid * _NC + cid
        if not write_m:
            acc = rest[0]
            pltpu.sync_copy(zeros_hbm.at[pl.ds(sid * _RPT, _RPT)],
                            acc.at[pl.ds(sid * _RPT, _RPT)])
            plsc.subcore_barrier()
        ebase = wid * _EW
        bufs = ((sd0, a0, b0, c0, sA0, sB0, sC0, sS0),
                (sd1, a1, b1, c1, sA1, sB1, sC1, sS1))

        def outdst(a, sd, ci):
            off = ebase + ci * _K2
            if write_m:
                return out_hbm.at[pl.ds(off, _K2)]
            return acc.at[sd.at[1]]

        def issue(ci, p, drain):
            sd, a, b, c, sa, sb, sc_, ss = bufs[p]
            off = ebase + ci * _K2

            def _drain():
                pltpu.make_async_copy(a, outdst(a, sd, ci - 2), ss).wait()

            if drain is True:
                _drain()
            elif drain is not False:
                pl.when(drain)(_drain)

            pltpu.sync_copy(sd_hbm.at[wid, ci], sd)
            pltpu.async_copy(hs_hbm.at[sd.at[0]], a, sa)
            pltpu.async_copy(hd_hbm.at[sd.at[1]], b, sb)
            pltpu.async_copy(rbf_hbm.at[pl.ds(off, _K2)], c, sc_)

        def finish(ci, p):
            sd, a, b, c, sa, sb, sc_, ss = bufs[p]
            off = ebase + ci * _K2
            pltpu.make_async_copy(hs_hbm.at[sd.at[0]], a, sa).wait()
            pltpu.make_async_copy(hd_hbm.at[sd.at[1]], b, sb).wait()
            pltpu.make_async_copy(rbf_hbm.at[pl.ds(off, _K2)], c, sc_).wait()

            def row(i, cr):
                for j in range(_H // _LANES):
                    sl = pl.ds(j * _LANES, _LANES)
                    x = a[i, sl] + b[i, sl] + c[i, sl]
                    a[i, sl] = jnp.maximum(x, 0.0)
                return cr

            lax.fori_loop(0, _K2, row, 0, unroll=2)
            if write_m:
                pltpu.async_copy(a, out_hbm.at[pl.ds(off, _K2)], ss)
            else:
                pltpu.async_copy(a, acc.at[sd.at[1]], ss, add=True)

        issue(0, 0, False)

        def pair(g, carry):
            ci0 = 2 * g
            issue(ci0 + 1, 1, g > 0)
            finish(ci0, 0)

            @pl.when(g < _NCHUNK2 // 2 - 1)
            def _():
                issue(ci0 + 2, 0, True)

            finish(ci0 + 1, 1)
            return carry

        lax.fori_loop(0, _NCHUNK2 // 2, pair, 0)
        for p, ci in ((0, _NCHUNK2 - 2), (1, _NCHUNK2 - 1)):
            sd, a, b, c, sa, sb, sc_, ss = bufs[p]
            pltpu.make_async_copy(a, outdst(a, sd, ci), ss).wait()
        if not write_m:
            plsc.subcore_barrier()
            pltpu.sync_copy(acc.at[pl.ds(sid * _RPT, _RPT)],
                            out_hbm.at[cid, pl.ds(sid * _RPT, _RPT)])

    if write_m:
        out_type = jax.ShapeDtypeStruct((_E, _H), _f32)
    else:
        out_type = jax.ShapeDtypeStruct((_NC, _N, _H), _f32)
    scratch = [
        pltpu.VMEM((2, _K2), jnp.int32),
        pltpu.VMEM((2, _K2), jnp.int32),
        pltpu.VMEM((_K2, _H), _f32),
        pltpu.VMEM((_K2, _H), _f32),
        pltpu.VMEM((_K2, _H), _f32),
        pltpu.VMEM((_K2, _H), _f32),
        pltpu.VMEM((_K2, _H), _f32),
        pltpu.VMEM((_K2, _H), _f32),
        pltpu.SemaphoreType.DMA,
        pltpu.SemaphoreType.DMA,
        pltpu.SemaphoreType.DMA,
        pltpu.SemaphoreType.DMA,
        pltpu.SemaphoreType.DMA,
        pltpu.SemaphoreType.DMA,
        pltpu.SemaphoreType.DMA,
        pltpu.SemaphoreType.DMA,
    ]
    if not write_m:
        scratch.append(pltpu.VMEM_SHARED((_N, _H), _f32))
    return pl.kernel(
        body,
        out_type=out_type,
        mesh=plsc.VectorSubcoreMesh(core_axis_name="c", subcore_axis_name="s", num_cores=_NC, num_subcores=_NS),
        compiler_params=pltpu.CompilerParams(use_tc_tiling_on_sc=False),
        scratch_types=scratch,
    )


def _fuscat_body(fu_hbm, sd_hbm, zeros_hbm, out_hbm, sd0, sd1, f0, f1,
                 sf0, sf1, ss0, ss1, acc):
    cid = lax.axis_index("c")
    sid = lax.axis_index("s")
    wid = sid * _NC + cid
    pltpu.sync_copy(zeros_hbm.at[pl.ds(sid * _RPT, _RPT)],
                    acc.at[pl.ds(sid * _RPT, _RPT)])
    plsc.subcore_barrier()
    ebase = wid * _EW
    bufs = ((sd0, f0, sf0, ss0), (sd1, f1, sf1, ss1))

    def issue(ci, p, drain):
        sd, f, sf, ss = bufs[p]

        def _drain():
            pltpu.make_async_copy(f, acc.at[sd.at[1]], ss).wait()

        if drain is True:
            _drain()
        elif drain is not False:
            pl.when(drain)(_drain)
        pltpu.sync_copy(sd_hbm.at[wid, ci], sd)
        pltpu.async_copy(fu_hbm.at[pl.ds(ebase + ci * _K2, _K2)], f, sf)

    def finish(ci, p):
        sd, f, sf, ss = bufs[p]
        pltpu.make_async_copy(fu_hbm.at[pl.ds(ebase + ci * _K2, _K2)], f, sf).wait()
        pltpu.async_copy(f, acc.at[sd.at[1]], ss, add=True)

    issue(0, 0, False)

    def pair(g, carry):
        ci0 = 2 * g
        issue(ci0 + 1, 1, g > 0)
        finish(ci0, 0)

        @pl.when(g < _NCHUNK2 // 2 - 1)
        def _():
            issue(ci0 + 2, 0, True)

        finish(ci0 + 1, 1)
        return carry

    lax.fori_loop(0, _NCHUNK2 // 2, pair, 0)
    for p in (0, 1):
        sd, f, sf, ss = bufs[p]
        pltpu.make_async_copy(f, acc.at[sd.at[1]], ss).wait()
    plsc.subcore_barrier()
    pltpu.sync_copy(acc.at[pl.ds(sid * _RPT, _RPT)],
                    out_hbm.at[cid, pl.ds(sid * _RPT, _RPT)])


def _fuscat_call_build():
  return pl.kernel(
    _fuscat_body,
    out_type=jax.ShapeDtypeStruct((_NC, _N, _LANES), _f32),
    mesh=plsc.VectorSubcoreMesh(core_axis_name="c", subcore_axis_name="s", num_cores=_NC, num_subcores=_NS),
    compiler_params=pltpu.CompilerParams(use_tc_tiling_on_sc=False),
    scratch_types=[
        pltpu.VMEM((2, _K2), jnp.int32),
        pltpu.VMEM((2, _K2), jnp.int32),
        pltpu.VMEM((_K2, _LANES), _f32),
        pltpu.VMEM((_K2, _LANES), _f32),
        pltpu.SemaphoreType.DMA,
        pltpu.SemaphoreType.DMA,
        pltpu.SemaphoreType.DMA,
        pltpu.SemaphoreType.DMA,
        pltpu.VMEM_SHARED((_N, _LANES), _f32),
    ],
  )


import functools as _functools


@_functools.lru_cache(maxsize=None)
def _sc_kernels():
    return (_vec_call_build(), _make_msgpass(False), _make_msgpass(True),
            _fuscat_call_build(), _embg_call_build())


def kernel(z, t, pred_frac_coords, pred_atom_types, num_atoms, lengths,
           angles, edge_index, emb_table, W_msg, b_msg, W_upd, b_upd, W_F,
           fc_atom_W, fc_atom_b):
    src = edge_index[0].astype(jnp.int32)
    dst = edge_index[1].astype(jnp.int32)
    Ws = W_msg[:_H]
    Wd = W_msg[_H:2 * _H]
    Wr = W_msg[2 * _H:]
    bm = b_msg.reshape(1, _H)
    bu = b_upd.reshape(1, _H)
    wf = W_F.reshape(1, _H)
    Wh = fc_atom_W[:_H]
    Wz = fc_atom_W[_H:]
    fb = fc_atom_b.reshape(1, _MAXZ)
    centers = jnp.linspace(0.0, _CUTOFF, _NRBF, dtype=_f32).reshape(1, _NRBF)
    types1 = pred_atom_types.astype(jnp.int32)
    zerosNH = jnp.zeros((_N, _H), _f32)
    sd2 = jnp.stack([src.reshape(_NW, _NCHUNK2, _K2),
                     dst.reshape(_NW, _NCHUNK2, _K2)], axis=2)
    zerosN16 = jnp.zeros((_N, _LANES), _f32)

    _vec_call, _msgpass_agg, _msgpass_m, _fuscat_call, _embg_call = _sc_kernels()
    cart16 = _cart_call(pred_frac_coords, lengths, angles)
    vec16 = _vec_call(cart16, sd2)
    unit16, rbfWb = _geom_call(vec16, Wr, bm, centers)
    h = _embg_call(emb_table, types1)
    hs, hd = _proj_call(h, Ws, Wd)
    for _ in range(2):
        agg2 = _msgpass_agg(hs, hd, rbfWb, sd2, zerosNH)
        h, hs, hd = _update_call(h, agg2[0], agg2[1], W_upd, bu, Ws, Wd)
    m = _msgpass_m(hs, hd, rbfWb, sd2, zerosNH)
    fu16 = _force_call(m, unit16, wf)
    fu2 = _fuscat_call(fu16, sd2, zerosN16)
    logits, diff16 = _final_call(h, z, Wh, Wz, fb, fu2[0], fu2[1])
    return diff16[:, :3], logits


# speedup vs baseline: 1.4749x; 1.4749x over previous
import jax
import jax.numpy as jnp
from jax import lax
from jax.experimental import pallas as pl
from jax.experimental.pallas import tpu as pltpu
from jax.experimental.pallas import tpu_sc as plsc

_N = 10000
_E = 320000
_H = 128
_LATENT = 256
_NRBF = 16
_MAXZ = 100
_CUTOFF = 6.0
_LANES = 16

_NC, _NS = 2, 16
_NW = _NC * _NS
_EW = _E // _NW
_K = 80
_NCHUNK = _EW // _K
_RPT = _N // _NS

_BN = 2000
_BE = 2560

_f32 = jnp.float32
_bf16 = jnp.bfloat16


def _dot_bf16(a, b):
    return jnp.dot(a.astype(_bf16), b.astype(_bf16), preferred_element_type=_f32)


def _cart_body(pf_ref, ln_ref, an_ref, out_ref):
    pf = pf_ref[...]
    f = pf - jnp.floor(pf)
    ln = ln_ref[...]
    rad = an_ref[...] * (jnp.pi / 180.0)
    ca = jnp.cos(rad[:, 0:1])
    cb = jnp.cos(rad[:, 1:2])
    cg = jnp.cos(rad[:, 2:3])
    sg = jnp.sin(rad[:, 2:3])
    a = ln[:, 0:1]
    b = ln[:, 1:2]
    c = ln[:, 2:3]
    cx = cb
    cy = (ca - cb * cg) / sg
    cz = jnp.sqrt(jnp.maximum(1.0 - cx * cx - cy * cy, 1e-6))
    tb = lambda x: x.astype(_bf16).astype(_f32)
    f0, f1, f2 = tb(f[:, 0:1]), tb(f[:, 1:2]), tb(f[:, 2:3])
    l00, l10, l20 = tb(a), tb(b * cg), tb(c * cx)
    l11, l21 = tb(b * sg), tb(c * cy)
    l22 = tb(c * cz)
    cartx = (f0 * l00 + f1 * l10) + f2 * l20
    carty = f1 * l11 + f2 * l21
    cartz = f2 * l22
    pad = jnp.zeros((pf.shape[0], _LANES - 3), _f32)
    out_ref[...] = jnp.concatenate([cartx, carty, cartz, pad], axis=1)


_cart_call = pl.pallas_call(
    _cart_body,
    grid=(_N // _BN,),
    in_specs=[pl.BlockSpec((_BN, 3), lambda i: (i, 0))] * 3,
    out_specs=pl.BlockSpec((_BN, _LANES), lambda i: (i, 0)),
    out_shape=jax.ShapeDtypeStruct((_N, _LANES), _f32),
)


def _geom_body(v_ref, wr_ref, bm_ref, cen_ref, u_ref, r_ref):
    v = v_ref[...]
    d2 = jnp.sum(v * v, axis=1, keepdims=True)
    dist = jnp.sqrt(d2) + 1e-8
    u_ref[...] = v / dist
    rbf = jnp.exp(-2.0 * (dist - cen_ref[...]) ** 2)
    r_ref[...] = _dot_bf16(rbf, wr_ref[...]) + bm_ref[...]


_geom_call = pl.pallas_call(
    _geom_body,
    grid=(_E // _BE,),
    in_specs=[
        pl.BlockSpec((_BE, _LANES), lambda i: (i, 0)),
        pl.BlockSpec((_NRBF, _H), lambda i: (0, 0)),
        pl.BlockSpec((1, _H), lambda i: (0, 0)),
        pl.BlockSpec((1, _NRBF), lambda i: (0, 0)),
    ],
    out_specs=[
        pl.BlockSpec((_BE, _LANES), lambda i: (i, 0)),
        pl.BlockSpec((_BE, _H), lambda i: (i, 0)),
    ],
    out_shape=[
        jax.ShapeDtypeStruct((_E, _LANES), _f32),
        jax.ShapeDtypeStruct((_E, _H), _f32),
    ],
)


def _proj_body(h_ref, ws_ref, wd_ref, hs_ref, hd_ref):
    h = h_ref[...]
    hs_ref[...] = _dot_bf16(h, ws_ref[...])
    hd_ref[...] = _dot_bf16(h, wd_ref[...])


_proj_call = pl.pallas_call(
    _proj_body,
    grid=(_N // _BN,),
    in_specs=[
        pl.BlockSpec((_BN, _H), lambda i: (i, 0)),
        pl.BlockSpec((_H, _H), lambda i: (0, 0)),
        pl.BlockSpec((_H, _H), lambda i: (0, 0)),
    ],
    out_specs=[pl.BlockSpec((_BN, _H), lambda i: (i, 0))] * 2,
    out_shape=[jax.ShapeDtypeStruct((_N, _H), _f32)] * 2,
)


def _update_body(h_ref, a0_ref, a1_ref, wu_ref, bu_ref, ws_ref, wd_ref,
                 hn_ref, hs_ref, hd_ref):
    agg = a0_ref[...] + a1_ref[...]
    hn = h_ref[...] + jnp.maximum(_dot_bf16(agg, wu_ref[...]) + bu_ref[...], 0.0)
    hn_ref[...] = hn
    hs_ref[...] = _dot_bf16(hn, ws_ref[...])
    hd_ref[...] = _dot_bf16(hn, wd_ref[...])


_update_call = pl.pallas_call(
    _update_body,
    grid=(_N // _BN,),
    in_specs=[
        pl.BlockSpec((_BN, _H), lambda i: (i, 0)),
        pl.BlockSpec((_BN, _H), lambda i: (i, 0)),
        pl.BlockSpec((_BN, _H), lambda i: (i, 0)),
        pl.BlockSpec((_H, _H), lambda i: (0, 0)),
        pl.BlockSpec((1, _H), lambda i: (0, 0)),
        pl.BlockSpec((_H, _H), lambda i: (0, 0)),
        pl.BlockSpec((_H, _H), lambda i: (0, 0)),
    ],
    out_specs=[pl.BlockSpec((_BN, _H), lambda i: (i, 0))] * 3,
    out_shape=[jax.ShapeDtypeStruct((_N, _H), _f32)] * 3,
)


def _force_body(m_ref, u_ref, wf_ref, fu_ref):
    mm = m_ref[...].astype(_bf16).astype(_f32)
    ww = wf_ref[...].astype(_bf16).astype(_f32)
    f = jnp.sum(mm * ww, axis=1, keepdims=True)
    fu_ref[...] = f * u_ref[...]


_force_call = pl.pallas_call(
    _force_body,
    grid=(_E // _BE,),
    in_specs=[
        pl.BlockSpec((_BE, _H), lambda i: (i, 0)),
        pl.BlockSpec((_BE, _LANES), lambda i: (i, 0)),
        pl.BlockSpec((1, _H), lambda i: (0, 0)),
    ],
    out_specs=pl.BlockSpec((_BE, _LANES), lambda i: (i, 0)),
    out_shape=jax.ShapeDtypeStruct((_E, _LANES), _f32),
)


def _final_body(h_ref, z_ref, wh_ref, wz_ref, b_ref, f0_ref, f1_ref,
                log_ref, d_ref):
    zz = z_ref[...]
    log_ref[...] = (
        (_dot_bf16(h_ref[...], wh_ref[...]) + _dot_bf16(zz[:, :_H], wz_ref[:_H, :]))
        + _dot_bf16(zz[:, _H:], wz_ref[_H:, :])
    ) + b_ref[...]
    d_ref[...] = f0_ref[...] + f1_ref[...]


_final_call = pl.pallas_call(
    _final_body,
    grid=(_N // _BN,),
    in_specs=[
        pl.BlockSpec((_BN, _H), lambda i: (i, 0)),
        pl.BlockSpec((_BN, _LATENT), lambda i: (i, 0)),
        pl.BlockSpec((_H, _MAXZ), lambda i: (0, 0)),
        pl.BlockSpec((_LATENT, _MAXZ), lambda i: (0, 0)),
        pl.BlockSpec((1, _MAXZ), lambda i: (0, 0)),
        pl.BlockSpec((_BN, _LANES), lambda i: (i, 0)),
        pl.BlockSpec((_BN, _LANES), lambda i: (i, 0)),
    ],
    out_specs=[
        pl.BlockSpec((_BN, _MAXZ), lambda i: (i, 0)),
        pl.BlockSpec((_BN, _LANES), lambda i: (i, 0)),
    ],
    out_shape=[
        jax.ShapeDtypeStruct((_N, _MAXZ), _f32),
        jax.ShapeDtypeStruct((_N, _LANES), _f32),
    ],
)


def _vec_body(cart_hbm, sd_hbm, out_hbm, sd0, sd1, a0, b0, a1, b1,
              sa0, sb0, sa1, sb1, ss0, ss1):
    cid = lax.axis_index("c")
    sid = lax.axis_index("s")
    wid = sid * _NC + cid
    ebase = wid * _EW
    bufs = ((sd0, a0, b0, sa0, sb0, ss0), (sd1, a1, b1, sa1, sb1, ss1))

    def issue(ci, p, drain):
        sd, a, b, sa, sb, ss = bufs[p]

        def _drain():
            pltpu.make_async_copy(a, out_hbm.at[pl.ds(ebase + (ci - 2) * _K2, _K2)], ss).wait()

        if drain is True:
            _drain()
        elif drain is not False:
            pl.when(drain)(_drain)
        pltpu.sync_copy(sd_hbm.at[wid, ci], sd)
        pltpu.async_copy(cart_hbm.at[sd.at[0]], a, sa)
        pltpu.async_copy(cart_hbm.at[sd.at[1]], b, sb)

    def finish(ci, p):
        sd, a, b, sa, sb, ss = bufs[p]
        pltpu.make_async_copy(cart_hbm.at[sd.at[0]], a, sa).wait()
        pltpu.make_async_copy(cart_hbm.at[sd.at[1]], b, sb).wait()

        def row(i, cr):
            a[i, :] = a[i, :] - b[i, :]
            return cr

        lax.fori_loop(0, _K2, row, 0)
        pltpu.async_copy(a, out_hbm.at[pl.ds(ebase + ci * _K2, _K2)], ss)

    issue(0, 0, False)

    def pair(g, carry):
        ci0 = 2 * g
        issue(ci0 + 1, 1, g > 0)
        finish(ci0, 0)

        @pl.when(g < _NCHUNK2 // 2 - 1)
        def _():
            issue(ci0 + 2, 0, True)

        finish(ci0 + 1, 1)
        return carry

    lax.fori_loop(0, _NCHUNK2 // 2, pair, 0)
    for p, ci in ((0, _NCHUNK2 - 2), (1, _NCHUNK2 - 1)):
        sd, a, b, sa, sb, ss = bufs[p]
        pltpu.make_async_copy(a, out_hbm.at[pl.ds(ebase + ci * _K2, _K2)], ss).wait()


def _vec_call_build():
  return pl.kernel(
    _vec_body,
    out_type=jax.ShapeDtypeStruct((_E, _LANES), _f32),
    mesh=plsc.VectorSubcoreMesh(core_axis_name="c", subcore_axis_name="s", num_cores=_NC, num_subcores=_NS),
    compiler_params=pltpu.CompilerParams(use_tc_tiling_on_sc=False),
    scratch_types=[
        pltpu.VMEM((2, _K2), jnp.int32),
        pltpu.VMEM((2, _K2), jnp.int32),
        pltpu.VMEM((_K2, _LANES), _f32),
        pltpu.VMEM((_K2, _LANES), _f32),
        pltpu.VMEM((_K2, _LANES), _f32),
        pltpu.VMEM((_K2, _LANES), _f32),
        pltpu.SemaphoreType.DMA,
        pltpu.SemaphoreType.DMA,
        pltpu.SemaphoreType.DMA,
        pltpu.SemaphoreType.DMA,
        pltpu.SemaphoreType.DMA,
        pltpu.SemaphoreType.DMA,
    ],
  )


_AW = 25
_AK = 80


def _embg_body(emb_hbm, t_hbm, out_hbm, tidx, rbuf, sg):
    cid = lax.axis_index("c")
    sid = lax.axis_index("s")
    wid = sid * _NC + cid
    base = wid * (_AK * 5)

    @pl.when(wid < _AW)
    def _():
        def chunk(ci, carry):
            off = base + ci * _AK
            pltpu.sync_copy(t_hbm.at[pl.ds(off, _AK)], tidx)
            pltpu.async_copy(emb_hbm.at[tidx], rbuf, sg).wait()
            pltpu.sync_copy(rbuf, out_hbm.at[pl.ds(off, _AK)])
            return carry

        lax.fori_loop(0, 5, chunk, 0)


def _embg_call_build():
    return pl.kernel(
        _embg_body,
        out_type=jax.ShapeDtypeStruct((_N, _H), _f32),
        mesh=plsc.VectorSubcoreMesh(core_axis_name="c", subcore_axis_name="s", num_cores=_NC, num_subcores=_NS),
        compiler_params=pltpu.CompilerParams(use_tc_tiling_on_sc=False),
        scratch_types=[
            pltpu.VMEM((_AK,), jnp.int32),
            pltpu.VMEM((_AK, _H), _f32),
            pltpu.SemaphoreType.DMA,
        ],
    )


_K2 = 40
_NCHUNK2 = _EW // _K2


def _make_msgpass(write_m):
    def body(hs_hbm, hd_hbm, rbf_hbm, sd_hbm, zeros_hbm, out_hbm,
             sd0, sd1, a0, b0, c0, a1, b1, c1, sA0, sB0, sC0, sA1,
             sB1, sC1, sS0, sS1, *rest):
        cid = lax.axis_index("c")
        sid = lax.axis_index("s")
        wid = sid * _NC + cid
        if not write_m:
            acc = rest[0]
            pltpu.sync_copy(zeros_hbm.at[pl.ds(sid * _RPT, _RPT)],
                            acc.at[pl.ds(sid * _RPT, _RPT)])
            plsc.subcore_barrier()
        ebase = wid * _EW
        bufs = ((sd0, a0, b0, c0, sA0, sB0, sC0, sS0),
                (sd1, a1, b1, c1, sA1, sB1, sC1, sS1))

        def outdst(a, sd, ci):
            off = ebase + ci * _K2
            if write_m:
                return out_hbm.at[pl.ds(off, _K2)]
            return acc.at[sd.at[1]]

        def issue(ci, p, drain):
            sd, a, b, c, sa, sb, sc_, ss = bufs[p]
            off = ebase + ci * _K2

            def _drain():
                pltpu.make_async_copy(a, outdst(a, sd, ci - 2), ss).wait()

            if drain is True:
                _drain()
            elif drain is not False:
                pl.when(drain)(_drain)

            pltpu.sync_copy(sd_hbm.at[wid, ci], sd)
            pltpu.async_copy(hs_hbm.at[sd.at[0]], a, sa)
            pltpu.async_copy(hd_hbm.at[sd.at[1]], b, sb)
            pltpu.async_copy(rbf_hbm.at[pl.ds(off, _K2)], c, sc_)

        def finish(ci, p):
            sd, a, b, c, sa, sb, sc_, ss = bufs[p]
            off = ebase + ci * _K2
            pltpu.make_async_copy(hs_hbm.at[sd.at[0]], a, sa).wait()
            pltpu.make_async_copy(hd_hbm.at[sd.at[1]], b, sb).wait()
            pltpu.make_async_copy(rbf_hbm.at[pl.ds(off, _K2)], c, sc_).wait()

            def row(i, cr):
                for j in range(_H // _LANES):
                    sl = pl.ds(j * _LANES, _LANES)
                    x = a[i, sl] + b[i, sl] + c[i, sl]
                    a[i, sl] = jnp.maximum(x, 0.0)
                return cr

            lax.fori_loop(0, _K2, row, 0)
            if write_m:
                pltpu.async_copy(a, out_hbm.at[pl.ds(off, _K2)], ss)
            else:
                pltpu.async_copy(a, acc.at[sd.at[1]], ss, add=True)

        issue(0, 0, False)

        def pair(g, carry):
            ci0 = 2 * g
            issue(ci0 + 1, 1, g > 0)
            finish(ci0, 0)

            @pl.when(g < _NCHUNK2 // 2 - 1)
            def _():
                issue(ci0 + 2, 0, True)

            finish(ci0 + 1, 1)
            return carry

        lax.fori_loop(0, _NCHUNK2 // 2, pair, 0)
        for p, ci in ((0, _NCHUNK2 - 2), (1, _NCHUNK2 - 1)):
            sd, a, b, c, sa, sb, sc_, ss = bufs[p]
            pltpu.make_async_copy(a, outdst(a, sd, ci), ss).wait()
        if not write_m:
            plsc.subcore_barrier()
            pltpu.sync_copy(acc.at[pl.ds(sid * _RPT, _RPT)],
                            out_hbm.at[cid, pl.ds(sid * _RPT, _RPT)])

    if write_m:
        out_type = jax.ShapeDtypeStruct((_E, _H), _f32)
    else:
        out_type = jax.ShapeDtypeStruct((_NC, _N, _H), _f32)
    scratch = [
        pltpu.VMEM((2, _K2), jnp.int32),
        pltpu.VMEM((2, _K2), jnp.int32),
        pltpu.VMEM((_K2, _H), _f32),
        pltpu.VMEM((_K2, _H), _f32),
        pltpu.VMEM((_K2, _H), _f32),
        pltpu.VMEM((_K2, _H), _f32),
        pltpu.VMEM((_K2, _H), _f32),
        pltpu.VMEM((_K2, _H), _f32),
        pltpu.SemaphoreType.DMA,
        pltpu.SemaphoreType.DMA,
        pltpu.SemaphoreType.DMA,
        pltpu.SemaphoreType.DMA,
        pltpu.SemaphoreType.DMA,
        pltpu.SemaphoreType.DMA,
        pltpu.SemaphoreType.DMA,
        pltpu.SemaphoreType.DMA,
    ]
    if not write_m:
        scratch.append(pltpu.VMEM_SHARED((_N, _H), _f32))
    return pl.kernel(
        body,
        out_type=out_type,
        mesh=plsc.VectorSubcoreMesh(core_axis_name="c", subcore_axis_name="s", num_cores=_NC, num_subcores=_NS),
        compiler_params=pltpu.CompilerParams(use_tc_tiling_on_sc=False),
        scratch_types=scratch,
    )


def _fuscat_body(fu_hbm, sd_hbm, zeros_hbm, out_hbm, sd0, sd1, f0, f1,
                 sf0, sf1, ss0, ss1, acc):
    cid = lax.axis_index("c")
    sid = lax.axis_index("s")
    wid = sid * _NC + cid
    pltpu.sync_copy(zeros_hbm.at[pl.ds(sid * _RPT, _RPT)],
                    acc.at[pl.ds(sid * _RPT, _RPT)])
    plsc.subcore_barrier()
    ebase = wid * _EW
    bufs = ((sd0, f0, sf0, ss0), (sd1, f1, sf1, ss1))

    def issue(ci, p, drain):
        sd, f, sf, ss = bufs[p]

        def _drain():
            pltpu.make_async_copy(f, acc.at[sd.at[1]], ss).wait()

        if drain is True:
            _drain()
        elif drain is not False:
            pl.when(drain)(_drain)
        pltpu.sync_copy(sd_hbm.at[wid, ci], sd)
        pltpu.async_copy(fu_hbm.at[pl.ds(ebase + ci * _K2, _K2)], f, sf)

    def finish(ci, p):
        sd, f, sf, ss = bufs[p]
        pltpu.make_async_copy(fu_hbm.at[pl.ds(ebase + ci * _K2, _K2)], f, sf).wait()
        pltpu.async_copy(f, acc.at[sd.at[1]], ss, add=True)

    issue(0, 0, False)

    def pair(g, carry):
        ci0 = 2 * g
        issue(ci0 + 1, 1, g > 0)
        finish(ci0, 0)

        @pl.when(g < _NCHUNK2 // 2 - 1)
        def _():
            issue(ci0 + 2, 0, True)

        finish(ci0 + 1, 1)
        return carry

    lax.fori_loop(0, _NCHUNK2 // 2, pair, 0)
    for p in (0, 1):
        sd, f, sf, ss = bufs[p]
        pltpu.make_async_copy(f, acc.at[sd.at[1]], ss).wait()
    plsc.subcore_barrier()
    pltpu.sync_copy(acc.at[pl.ds(sid * _RPT, _RPT)],
                    out_hbm.at[cid, pl.ds(sid * _RPT, _RPT)])


def _fuscat_call_build():
  return pl.kernel(
    _fuscat_body,
    out_type=jax.ShapeDtypeStruct((_NC, _N, _LANES), _f32),
    mesh=plsc.VectorSubcoreMesh(core_axis_name="c", subcore_axis_name="s", num_cores=_NC, num_subcores=_NS),
    compiler_params=pltpu.CompilerParams(use_tc_tiling_on_sc=False),
    scratch_types=[
        pltpu.VMEM((2, _K2), jnp.int32),
        pltpu.VMEM((2, _K2), jnp.int32),
        pltpu.VMEM((_K2, _LANES), _f32),
        pltpu.VMEM((_K2, _LANES), _f32),
        pltpu.SemaphoreType.DMA,
        pltpu.SemaphoreType.DMA,
        pltpu.SemaphoreType.DMA,
        pltpu.SemaphoreType.DMA,
        pltpu.VMEM_SHARED((_N, _LANES), _f32),
    ],
  )


import functools as _functools


@_functools.lru_cache(maxsize=None)
def _sc_kernels():
    return (_vec_call_build(), _make_msgpass(False), _make_msgpass(True),
            _fuscat_call_build(), _embg_call_build())


def kernel(z, t, pred_frac_coords, pred_atom_types, num_atoms, lengths,
           angles, edge_index, emb_table, W_msg, b_msg, W_upd, b_upd, W_F,
           fc_atom_W, fc_atom_b):
    src = edge_index[0].astype(jnp.int32)
    dst = edge_index[1].astype(jnp.int32)
    Ws = W_msg[:_H]
    Wd = W_msg[_H:2 * _H]
    Wr = W_msg[2 * _H:]
    bm = b_msg.reshape(1, _H)
    bu = b_upd.reshape(1, _H)
    wf = W_F.reshape(1, _H)
    Wh = fc_atom_W[:_H]
    Wz = fc_atom_W[_H:]
    fb = fc_atom_b.reshape(1, _MAXZ)
    centers = jnp.linspace(0.0, _CUTOFF, _NRBF, dtype=_f32).reshape(1, _NRBF)
    types1 = pred_atom_types.astype(jnp.int32)
    zerosNH = jnp.zeros((_N, _H), _f32)
    sd2 = jnp.stack([src.reshape(_NW, _NCHUNK2, _K2),
                     dst.reshape(_NW, _NCHUNK2, _K2)], axis=2)
    zerosN16 = jnp.zeros((_N, _LANES), _f32)

    _vec_call, _msgpass_agg, _msgpass_m, _fuscat_call, _embg_call = _sc_kernels()
    cart16 = _cart_call(pred_frac_coords, lengths, angles)
    vec16 = _vec_call(cart16, sd2)
    unit16, rbfWb = _geom_call(vec16, Wr, bm, centers)
    h = _embg_call(emb_table, types1)
    hs, hd = _proj_call(h, Ws, Wd)
    for _ in range(2):
        agg2 = _msgpass_agg(hs, hd, rbfWb, sd2, zerosNH)
        h, hs, hd = _update_call(h, agg2[0], agg2[1], W_upd, bu, Ws, Wd)
    m = _msgpass_m(hs, hd, rbfWb, sd2, zerosNH)
    fu16 = _force_call(m, unit16, wf)
    fu2 = _fuscat_call(fu16, sd2, zerosN16)
    logits, diff16 = _final_call(h, z, Wh, Wz, fb, fu2[0], fu2[1])
    return diff16[:, :3], logits
